# Initial kernel scaffold; baseline (speedup 1.0000x reference)
import jax
import jax.numpy as jnp
from jax.experimental import pallas as pl


def kernel(R, t, res_feat, pair_feat, mask_res, params):
    raise NotImplementedError("write your pallas kernel here")


# speedup vs baseline: 9.5852x; 9.5852x over previous
import functools

import jax
import jax.numpy as jnp
from jax import lax
from jax.experimental import pallas as pl
from jax.experimental.pallas import tpu as pltpu
from jax.experimental.pallas import tpu_sc as plsc

_B, _L, _K = 4, 1024, 30
_D = 128
_N = _B * _L
_E = _N * _K


_RB = 256


def _knn_body(pos_r_ref, pos_j_ref, pairidx_ref, rdsum_ref):
    b = pl.program_id(0)
    r = pl.program_id(1)
    pr = pos_r_ref[0]
    pj = pos_j_ref[0]
    acc = (pr[:, 0:1] - pj[0:1, :]) ** 2
    acc = acc + (pr[:, 1:2] - pj[1:2, :]) ** 2
    acc = acc + (pr[:, 2:3] - pj[2:3, :]) ** 2
    jio = lax.broadcasted_iota(jnp.int32, (_RB, _L), 1)
    rowg = lax.broadcasted_iota(jnp.int32, (_RB, _L), 0) + r * _RB
    d2 = jnp.where(jio == rowg, jnp.float32(1e9), acc)
    cols = []
    rdsum = jnp.zeros((_RB, 1), jnp.float32)
    for _ in range(_K):
        m = jnp.min(d2, axis=1, keepdims=True)
        idx = jnp.min(jnp.where(d2 == m, jio, _L), axis=1, keepdims=True)
        d2 = jnp.where(jio == idx, jnp.float32(jnp.inf), d2)
        rdsum = rdsum + jnp.sqrt(m)
        cols.append(idx)
    nbr = jnp.concatenate(cols, axis=1)
    ctr = lax.broadcasted_iota(jnp.int32, (_RB, _K), 0) + r * _RB
    pairidx_ref[0] = b * (_L * _L) + nbr * _L + ctr
    rdsum_ref[0] = rdsum


def _knn(pos, pos_t):
    return pl.pallas_call(
        _knn_body,
        grid=(_B, _L // _RB),
        in_specs=[
            pl.BlockSpec((1, _RB, 3), lambda b, r: (b, r, 0)),
            pl.BlockSpec((1, 3, _L), lambda b, r: (b, 0, 0)),
        ],
        out_specs=[
            pl.BlockSpec((1, _RB, _K), lambda b, r: (b, r, 0)),
            pl.BlockSpec((1, _RB, 1), lambda b, r: (b, r, 0)),
        ],
        out_shape=[
            jax.ShapeDtypeStruct((_B, _L, _K), jnp.int32),
            jax.ShapeDtypeStruct((_B, _L, 1), jnp.float32),
        ],
    )(pos, pos_t)


_NC, _NS = 2, 16
_NW = _NC * _NS
_NODES_PER = _N // _NW
_CHUNKS = _NODES_PER * _K // 128


_EPT = _NODES_PER * _K
_QTOT = 4 * _CHUNKS
_QBATCH = 24


def _gather_body(table_hbm, idx_hbm, out_hbm, idx_v, ch_v, agg_v, sem):
    wid = lax.axis_index("s") * _NC + lax.axis_index("c")
    for c in range(4):
        pltpu.sync_copy(
            idx_hbm.at[pl.ds(c * (_E // 128) + wid * _CHUNKS, _CHUNKS)],
            idx_v.at[pl.ds(c * _CHUNKS, _CHUNKS)])

    def fire(q):
        pltpu.async_copy(
            table_hbm.at[idx_v.at[q]], ch_v.at[pl.ds(q * 128, 128)], sem)

    def drain(q):
        pltpu.make_async_copy(
            table_hbm.at[idx_v.at[q]], ch_v.at[pl.ds(q * 128, 128)], sem).wait()

    nb = _QTOT // _QBATCH
    for i in range(nb):
        for q in range(i * _QBATCH, (i + 1) * _QBATCH):
            fire(q)
        if i > 0:
            for q in range((i - 1) * _QBATCH, i * _QBATCH):
                drain(q)
    for q in range((nb - 1) * _QBATCH, nb * _QBATCH):
        drain(q)

    lane = lax.broadcasted_iota(jnp.int32, (16,), 0)
    inv_k = jnp.full((16,), 1.0 / _K, jnp.float32)

    def body(g, carry):
        nbase = (g * 16 + lane) * _K
        for c in range(4):
            acc = jnp.zeros((16,), jnp.float32)
            for k in range(_K):
                acc = acc + plsc.load_gather(ch_v, [nbase + (c * _EPT + k)])
            agg_v[pl.ds(c * _NODES_PER + g * 16, 16)] = acc * inv_k
        return carry

    lax.fori_loop(0, _NODES_PER // 16, body, 0)
    for c in range(4):
        pltpu.sync_copy(
            agg_v.at[pl.ds(c * _NODES_PER, _NODES_PER)],
            out_hbm.at[pl.ds(c * _N + wid * _NODES_PER, _NODES_PER)])


@functools.cache
def _gather_call():
    return pl.kernel(
        _gather_body,
        out_type=jax.ShapeDtypeStruct((4 * _N,), jnp.float32),
        mesh=plsc.VectorSubcoreMesh(
            core_axis_name="c", subcore_axis_name="s",
            num_cores=_NC, num_subcores=_NS,
        ),
        scratch_types=[
            pltpu.VMEM((_QTOT, 128), jnp.int32),
            pltpu.VMEM((_QTOT * 128,), jnp.float32),
            pltpu.VMEM((4 * _NODES_PER,), jnp.float32),
            pltpu.SemaphoreType.DMA,
        ],
        compiler_params=pltpu.CompilerParams(
            use_tc_tiling_on_sc=False, needs_layout_passes=False),
    )


_RM = 512


def _mlp_body(x_ref, ap_ref, rs_ref, w1x_ref, w1p_ref, w1r_ref,
              b1_ref, w2_ref, b2_ref, out_ref):
    x = x_ref[...]
    ap = ap_ref[...]
    rs = rs_ref[...] * jnp.float32(1.0 / _K)
    hp = jax.lax.Precision.HIGHEST
    for l in range(3):
        h = jnp.dot(x, w1x_ref[l], preferred_element_type=jnp.float32,
                    precision=hp)
        for c in range(4):
            h = h + ap[:, c:c + 1] * w1p_ref[l, c:c + 1, :]
        h = h + rs * w1r_ref[l] + b1_ref[l]
        h = jnp.maximum(h, 0.0)
        x = jnp.dot(h, w2_ref[l], preferred_element_type=jnp.float32,
                    precision=hp) + b2_ref[l]
    out_ref[...] = x


def _mlp(x, aggp, rdsum, w1x, w1p, w1r, b1, w2, b2):
    full = lambda shape: pl.BlockSpec(shape, lambda i: tuple(0 for _ in shape))
    return pl.pallas_call(
        _mlp_body,
        grid=(_N // _RM,),
        in_specs=[
            pl.BlockSpec((_RM, _D), lambda i: (i, 0)),
            pl.BlockSpec((_RM, 4), lambda i: (i, 0)),
            pl.BlockSpec((_RM, 1), lambda i: (i, 0)),
            full((3, _D, _D)),
            full((3, 4, _D)),
            full((3, 1, _D)),
            full((3, 1, _D)),
            full((3, _D, _D)),
            full((3, 1, _D)),
        ],
        out_specs=pl.BlockSpec((_RM, _D), lambda i: (i, 0)),
        out_shape=jax.ShapeDtypeStruct((_N, _D), jnp.float32),
    )(x, aggp, rdsum, w1x, w1p, w1r, b1, w2, b2)


def kernel(R, t, res_feat, pair_feat, mask_res, params):
    pos = t
    pos_t = jnp.swapaxes(t, 1, 2)
    pairidx, rdsum = _knn(pos, pos_t)
    tflat = pair_feat.transpose(3, 0, 1, 2).reshape(4 * _B * _L * _L)
    idx4 = (pairidx.reshape(-1)[None, :]
            + (jnp.arange(4, dtype=jnp.int32) * (_B * _L * _L))[:, None])
    idx4 = idx4.reshape(4 * (_E // 128), 128)
    aggp = _gather_call()(tflat, idx4).reshape(4, _N).T
    w1x = jnp.stack([p["nw1"][:_D] for p in params])
    w1p = jnp.stack([p["nw1"][_D:_D + 4] for p in params])
    w1r = jnp.stack([p["nw1"][_D + 4:_D + 5] for p in params])
    b1 = jnp.stack([p["nb1"][None, :] for p in params])
    w2 = jnp.stack([p["nw2"] for p in params])
    b2 = jnp.stack([p["nb2"][None, :] for p in params])
    x = res_feat.reshape(_N, _D)
    out = _mlp(x, aggp, rdsum.reshape(_N, 1), w1x, w1p, w1r, b1, w2, b2)
    return out.reshape(_B, _L, _D)


# speedup vs baseline: 11.5007x; 1.1998x over previous
import functools

import jax
import jax.numpy as jnp
from jax import lax
from jax.experimental import pallas as pl
from jax.experimental.pallas import tpu as pltpu
from jax.experimental.pallas import tpu_sc as plsc

_B, _L, _K = 4, 1024, 30
_D = 128
_N = _B * _L
_E = _N * _K


_RB = 256


def _knn_body(pos_r_ref, pos_j_ref, pairidx_ref, rdsum_ref):
    b = pl.program_id(0)
    r = pl.program_id(1)
    pr = pos_r_ref[0]
    pj = pos_j_ref[0]
    acc = (pr[:, 0:1] - pj[0:1, :]) ** 2
    acc = acc + (pr[:, 1:2] - pj[1:2, :]) ** 2
    acc = acc + (pr[:, 2:3] - pj[2:3, :]) ** 2
    jio = lax.broadcasted_iota(jnp.int32, (_RB, _L), 1)
    rowg = lax.broadcasted_iota(jnp.int32, (_RB, _L), 0) + r * _RB
    d2 = jnp.where(jio == rowg, jnp.float32(1e9), acc)
    cols = []
    rdsum = jnp.zeros((_RB, 1), jnp.float32)
    for _ in range(_K):
        m = jnp.min(d2, axis=1, keepdims=True)
        idx = jnp.min(jnp.where(d2 == m, jio, _L), axis=1, keepdims=True)
        d2 = jnp.where(jio == idx, jnp.float32(jnp.inf), d2)
        rdsum = rdsum + jnp.sqrt(m)
        cols.append(idx)
    nbr = jnp.concatenate(cols, axis=1)
    ctr = lax.broadcasted_iota(jnp.int32, (_RB, _K), 0) + r * _RB
    pairidx_ref[0] = (b * (_L * 4096) + nbr * 4096
                      + (ctr >> 7) * 512 + (ctr & 127))
    rdsum_ref[0] = rdsum


def _knn(pos, pos_t):
    return pl.pallas_call(
        _knn_body,
        grid=(_B, _L // _RB),
        in_specs=[
            pl.BlockSpec((1, _RB, 3), lambda b, r: (b, r, 0)),
            pl.BlockSpec((1, 3, _L), lambda b, r: (b, 0, 0)),
        ],
        out_specs=[
            pl.BlockSpec((1, _RB, _K), lambda b, r: (b, r, 0)),
            pl.BlockSpec((1, _RB, 1), lambda b, r: (b, r, 0)),
        ],
        out_shape=[
            jax.ShapeDtypeStruct((_B, _L, _K), jnp.int32),
            jax.ShapeDtypeStruct((_B, _L, 1), jnp.float32),
        ],
    )(pos, pos_t)


_NC, _NS = 2, 16
_NW = _NC * _NS
_NODES_PER = _N // _NW
_CHUNKS = _NODES_PER * _K // 128


_EPT = _NODES_PER * _K
_QTOT = 4 * _CHUNKS
_QBATCH = 24


def _gather_body(table_hbm, idx_hbm, out_hbm, idx_v, ch_v, agg_v, sem):
    wid = lax.axis_index("s") * _NC + lax.axis_index("c")
    for c in range(4):
        pltpu.sync_copy(
            idx_hbm.at[pl.ds(c * (_E // 128) + wid * _CHUNKS, _CHUNKS)],
            idx_v.at[pl.ds(c * _CHUNKS, _CHUNKS)])

    def fire(q):
        pltpu.async_copy(
            table_hbm.at[idx_v.at[q]], ch_v.at[pl.ds(q * 128, 128)], sem)

    def drain(q):
        pltpu.make_async_copy(
            table_hbm.at[idx_v.at[q]], ch_v.at[pl.ds(q * 128, 128)], sem).wait()

    nb = _QTOT // _QBATCH
    for i in range(nb):
        for q in range(i * _QBATCH, (i + 1) * _QBATCH):
            fire(q)
        if i > 0:
            for q in range((i - 1) * _QBATCH, i * _QBATCH):
                drain(q)
    for q in range((nb - 1) * _QBATCH, nb * _QBATCH):
        drain(q)

    lane = lax.broadcasted_iota(jnp.int32, (16,), 0)
    inv_k = jnp.full((16,), 1.0 / _K, jnp.float32)

    def body(g, carry):
        nbase = (g * 16 + lane) * _K
        for c in range(4):
            acc = jnp.zeros((16,), jnp.float32)
            for k in range(_K):
                acc = acc + plsc.load_gather(ch_v, [nbase + (c * _EPT + k)])
            agg_v[pl.ds(c * _NODES_PER + g * 16, 16)] = acc * inv_k
        return carry

    lax.fori_loop(0, _NODES_PER // 16, body, 0)
    for c in range(4):
        pltpu.sync_copy(
            agg_v.at[pl.ds(c * _NODES_PER, _NODES_PER)],
            out_hbm.at[pl.ds(c * _N + wid * _NODES_PER, _NODES_PER)])


@functools.cache
def _gather_call():
    return pl.kernel(
        _gather_body,
        out_type=jax.ShapeDtypeStruct((4 * _N,), jnp.float32),
        mesh=plsc.VectorSubcoreMesh(
            core_axis_name="c", subcore_axis_name="s",
            num_cores=_NC, num_subcores=_NS,
        ),
        scratch_types=[
            pltpu.VMEM((_QTOT, 128), jnp.int32),
            pltpu.VMEM((_QTOT * 128,), jnp.float32),
            pltpu.VMEM((4 * _NODES_PER,), jnp.float32),
            pltpu.SemaphoreType.DMA,
        ],
        compiler_params=pltpu.CompilerParams(
            use_tc_tiling_on_sc=False, needs_layout_passes=False),
    )


_RM = 512


def _mlp_body(x_ref, ap_ref, rs_ref, w1x_ref, w1p_ref, w1r_ref,
              b1_ref, w2_ref, b2_ref, out_ref):
    x = x_ref[...]
    ap = ap_ref[...]
    rs = rs_ref[...] * jnp.float32(1.0 / _K)
    hp = jax.lax.Precision.HIGHEST
    for l in range(3):
        h = jnp.dot(x, w1x_ref[l], preferred_element_type=jnp.float32,
                    precision=hp)
        for c in range(4):
            h = h + ap[:, c:c + 1] * w1p_ref[l, c:c + 1, :]
        h = h + rs * w1r_ref[l] + b1_ref[l]
        h = jnp.maximum(h, 0.0)
        x = jnp.dot(h, w2_ref[l], preferred_element_type=jnp.float32,
                    precision=hp) + b2_ref[l]
    out_ref[...] = x


def _mlp(x, aggp, rdsum, w1x, w1p, w1r, b1, w2, b2):
    full = lambda shape: pl.BlockSpec(shape, lambda i: tuple(0 for _ in shape))
    return pl.pallas_call(
        _mlp_body,
        grid=(_N // _RM,),
        in_specs=[
            pl.BlockSpec((_RM, _D), lambda i: (i, 0)),
            pl.BlockSpec((_RM, 4), lambda i: (i, 0)),
            pl.BlockSpec((_RM, 1), lambda i: (i, 0)),
            full((3, _D, _D)),
            full((3, 4, _D)),
            full((3, 1, _D)),
            full((3, 1, _D)),
            full((3, _D, _D)),
            full((3, 1, _D)),
        ],
        out_specs=pl.BlockSpec((_RM, _D), lambda i: (i, 0)),
        out_shape=jax.ShapeDtypeStruct((_N, _D), jnp.float32),
    )(x, aggp, rdsum, w1x, w1p, w1r, b1, w2, b2)


def kernel(R, t, res_feat, pair_feat, mask_res, params):
    pos = t
    pos_t = jnp.swapaxes(t, 1, 2)
    pairidx, rdsum = _knn(pos, pos_t)
    tflat = (pair_feat.reshape(_B, _L, _L // 128, 128, 4)
             .transpose(0, 1, 2, 4, 3).reshape(4 * _B * _L * _L))
    idx4 = (pairidx.reshape(-1)[None, :]
            + (jnp.arange(4, dtype=jnp.int32) * 128)[:, None])
    idx4 = idx4.reshape(4 * (_E // 128), 128)
    aggp = _gather_call()(tflat, idx4).reshape(4, _N).T
    w1x = jnp.stack([p["nw1"][:_D] for p in params])
    w1p = jnp.stack([p["nw1"][_D:_D + 4] for p in params])
    w1r = jnp.stack([p["nw1"][_D + 4:_D + 5] for p in params])
    b1 = jnp.stack([p["nb1"][None, :] for p in params])
    w2 = jnp.stack([p["nw2"] for p in params])
    b2 = jnp.stack([p["nb2"][None, :] for p in params])
    x = res_feat.reshape(_N, _D)
    out = _mlp(x, aggp, rdsum.reshape(_N, 1), w1x, w1p, w1r, b1, w2, b2)
    return out.reshape(_B, _L, _D)


# speedup vs baseline: 15.5148x; 1.3490x over previous
import functools

import jax
import jax.numpy as jnp
from jax import lax
from jax.experimental import pallas as pl
from jax.experimental.pallas import tpu as pltpu
from jax.experimental.pallas import tpu_sc as plsc

_B, _L, _K = 4, 1024, 30
_D = 128
_N = _B * _L
_E = _N * _K


_RB = 256


_NCH = 8


def _knn_body(pos_r_ref, pos_j_ref, pairidx_ref, rdsum_ref):
    b = pl.program_id(0)
    r = pl.program_id(1)
    pr = pos_r_ref[0]
    pj = pos_j_ref[0]
    jiof = lax.broadcasted_iota(jnp.int32, (_RB, _L), 1).astype(jnp.float32)
    rowgf = (lax.broadcasted_iota(jnp.int32, (_RB, _L), 0) + r * _RB
             ).astype(jnp.float32)
    acc = (pr[:, 0:1] - pj[0:1, :]) ** 2
    acc = acc + (pr[:, 1:2] - pj[1:2, :]) ** 2
    acc = acc + (pr[:, 2:3] - pj[2:3, :]) ** 2
    d2 = jnp.where(jiof == rowgf, jnp.float32(1e9), acc)
    big = jnp.float32(3e38)
    cols = []
    rdsum = jnp.zeros((_RB, 1), jnp.float32)
    idxf = None
    for _ in range(_K):
        if idxf is not None:
            d2 = jnp.where(jiof == idxf, big, d2)
        m = jnp.min(d2, axis=1, keepdims=True)
        idxf = jnp.min(jnp.where(d2 == m, jiof, jnp.float32(4096.0)),
                       axis=1, keepdims=True)
        rdsum = rdsum + jnp.sqrt(m)
        cols.append(idxf)
    nbr = jnp.concatenate(cols, axis=1).astype(jnp.int32)
    ctr = lax.broadcasted_iota(jnp.int32, (_RB, _K), 0) + r * _RB
    pairidx_ref[0] = (b * (_L * 4096) + nbr * 4096
                      + (ctr >> 7) * 512 + (ctr & 127))
    rdsum_ref[0] = rdsum


def _knn(pos, pos_t):
    return pl.pallas_call(
        _knn_body,
        grid=(_B, _L // _RB),
        in_specs=[
            pl.BlockSpec((1, _RB, 3), lambda b, r: (b, r, 0)),
            pl.BlockSpec((1, 3, _L), lambda b, r: (b, 0, 0)),
        ],
        out_specs=[
            pl.BlockSpec((1, _RB, _K), lambda b, r: (b, r, 0)),
            pl.BlockSpec((1, _RB, 1), lambda b, r: (b, r, 0)),
        ],
        out_shape=[
            jax.ShapeDtypeStruct((_B, _L, _K), jnp.int32),
            jax.ShapeDtypeStruct((_B, _L, 1), jnp.float32),
        ],
    )(pos, pos_t)


_NC, _NS = 2, 16
_NW = _NC * _NS
_NODES_PER = _N // _NW
_CHUNKS = _NODES_PER * _K // 128


_EPT = _NODES_PER * _K
_QTOT = 4 * _CHUNKS
_QBATCH = 24


def _gather_body(table_hbm, idx_hbm, out_hbm, idx_v, ch_v, agg_v, sem):
    wid = lax.axis_index("s") * _NC + lax.axis_index("c")
    for c in range(4):
        pltpu.sync_copy(
            idx_hbm.at[pl.ds(c * (_E // 128) + wid * _CHUNKS, _CHUNKS)],
            idx_v.at[pl.ds(c * _CHUNKS, _CHUNKS)])

    def fire(q):
        pltpu.async_copy(
            table_hbm.at[idx_v.at[q]], ch_v.at[pl.ds(q * 128, 128)], sem)

    def drain(q):
        pltpu.make_async_copy(
            table_hbm.at[idx_v.at[q]], ch_v.at[pl.ds(q * 128, 128)], sem).wait()

    nb = _QTOT // _QBATCH
    for i in range(nb):
        for q in range(i * _QBATCH, (i + 1) * _QBATCH):
            fire(q)
        if i > 0:
            for q in range((i - 1) * _QBATCH, i * _QBATCH):
                drain(q)
    for q in range((nb - 1) * _QBATCH, nb * _QBATCH):
        drain(q)

    lane = lax.broadcasted_iota(jnp.int32, (16,), 0)
    inv_k = jnp.full((16,), 1.0 / _K, jnp.float32)

    def body(g, carry):
        nbase = (g * 16 + lane) * _K
        for c in range(4):
            acc = jnp.zeros((16,), jnp.float32)
            for k in range(_K):
                acc = acc + plsc.load_gather(ch_v, [nbase + (c * _EPT + k)])
            agg_v[pl.ds(c * _NODES_PER + g * 16, 16)] = acc * inv_k
        return carry

    lax.fori_loop(0, _NODES_PER // 16, body, 0)
    for c in range(4):
        pltpu.sync_copy(
            agg_v.at[pl.ds(c * _NODES_PER, _NODES_PER)],
            out_hbm.at[pl.ds(c * _N + wid * _NODES_PER, _NODES_PER)])


@functools.cache
def _gather_call():
    return pl.kernel(
        _gather_body,
        out_type=jax.ShapeDtypeStruct((4 * _N,), jnp.float32),
        mesh=plsc.VectorSubcoreMesh(
            core_axis_name="c", subcore_axis_name="s",
            num_cores=_NC, num_subcores=_NS,
        ),
        scratch_types=[
            pltpu.VMEM((_QTOT, 128), jnp.int32),
            pltpu.VMEM((_QTOT * 128,), jnp.float32),
            pltpu.VMEM((4 * _NODES_PER,), jnp.float32),
            pltpu.SemaphoreType.DMA,
        ],
        compiler_params=pltpu.CompilerParams(
            use_tc_tiling_on_sc=False, needs_layout_passes=False),
    )


_RM = 512


def _mlp_body(x_ref, ap_ref, rs_ref, *refs):
    w1s, b1s, w2s, b2s = refs[0:3], refs[3:6], refs[6:9], refs[9:12]
    out_ref = refs[12]
    x = x_ref[...]
    ap = ap_ref[...]
    rs = rs_ref[...] * jnp.float32(1.0 / _K)
    for l in range(3):
        w1 = w1s[l]
        h = jnp.dot(x, w1[0:_D, :], preferred_element_type=jnp.float32)
        for c in range(4):
            h = h + ap[:, c:c + 1] * w1[_D + c:_D + c + 1, :]
        h = h + rs * w1[_D + 4:_D + 5, :] + b1s[l][...]
        h = jnp.maximum(h, 0.0)
        x = jnp.dot(h, w2s[l][...], preferred_element_type=jnp.float32) + b2s[l][...]
    out_ref[...] = x


def _mlp(x, aggp, rdsum, w1s, b1s, w2s, b2s):
    full = lambda shape: pl.BlockSpec(shape, lambda i: tuple(0 for _ in shape))
    return pl.pallas_call(
        _mlp_body,
        grid=(_N // _RM,),
        in_specs=[
            pl.BlockSpec((_RM, _D), lambda i: (i, 0)),
            pl.BlockSpec((_RM, 4), lambda i: (i, 0)),
            pl.BlockSpec((_RM, 1), lambda i: (i, 0)),
            *[full((_D + 5, _D)) for _ in range(3)],
            *[full((1, _D)) for _ in range(3)],
            *[full((_D, _D)) for _ in range(3)],
            *[full((1, _D)) for _ in range(3)],
        ],
        out_specs=pl.BlockSpec((_RM, _D), lambda i: (i, 0)),
        out_shape=jax.ShapeDtypeStruct((_N, _D), jnp.float32),
    )(x, aggp, rdsum, *w1s, *b1s, *w2s, *b2s)


def kernel(R, t, res_feat, pair_feat, mask_res, params):
    pos = t
    pos_t = jnp.swapaxes(t, 1, 2)
    pairidx, rdsum = _knn(pos, pos_t)
    tflat = (pair_feat.reshape(_B, _L, _L // 128, 128, 4)
             .transpose(0, 1, 2, 4, 3).reshape(4 * _B * _L * _L))
    idx4 = (pairidx.reshape(-1)[None, :]
            + (jnp.arange(4, dtype=jnp.int32) * 128)[:, None])
    idx4 = idx4.reshape(4 * (_E // 128), 128)
    aggp = _gather_call()(tflat, idx4).reshape(4, _N).T
    w1s = [p["nw1"] for p in params]
    b1s = [p["nb1"].reshape(1, _D) for p in params]
    w2s = [p["nw2"] for p in params]
    b2s = [p["nb2"].reshape(1, _D) for p in params]
    x = res_feat.reshape(_N, _D)
    out = _mlp(x, aggp, rdsum.reshape(_N, 1), w1s, b1s, w2s, b2s)
    return out.reshape(_B, _L, _D)


# speedup vs baseline: 16.2025x; 1.0443x over previous
import functools

import jax
import jax.numpy as jnp
from jax import lax
from jax.experimental import pallas as pl
from jax.experimental.pallas import tpu as pltpu
from jax.experimental.pallas import tpu_sc as plsc

_B, _L, _K = 4, 1024, 30
_D = 128
_N = _B * _L
_E = _N * _K


_RB = 256


_NCH = 8


def _knn_body(pos_r_ref, pos_j_ref, pairidx_ref, rdsum_ref):
    b = pl.program_id(0)
    r = pl.program_id(1)
    pr = pos_r_ref[0]
    pj = pos_j_ref[0]
    jiof = lax.broadcasted_iota(jnp.int32, (_RB, _L), 1).astype(jnp.float32)
    rowgf = (lax.broadcasted_iota(jnp.int32, (_RB, _L), 0) + r * _RB
             ).astype(jnp.float32)
    acc = (pr[:, 0:1] - pj[0:1, :]) ** 2
    acc = acc + (pr[:, 1:2] - pj[1:2, :]) ** 2
    acc = acc + (pr[:, 2:3] - pj[2:3, :]) ** 2
    d2 = jnp.where(jiof == rowgf, jnp.float32(1e9), acc)
    big = jnp.float32(3e38)
    cols = []
    rdsum = jnp.zeros((_RB, 1), jnp.float32)
    idxf = None
    for _ in range(_K):
        if idxf is not None:
            d2 = jnp.where(jiof == idxf, big, d2)
        m = jnp.min(d2, axis=1, keepdims=True)
        idxf = jnp.min(jnp.where(d2 == m, jiof, jnp.float32(4096.0)),
                       axis=1, keepdims=True)
        rdsum = rdsum + jnp.sqrt(m)
        cols.append(idxf)
    nbr = jnp.concatenate(cols, axis=1).astype(jnp.int32)
    nbr_t = jnp.transpose(nbr)
    nbr_t = jnp.concatenate([nbr_t, nbr_t[0:2]], axis=0)
    ctr = lax.broadcasted_iota(jnp.int32, (32, _RB), 1) + r * _RB
    pairidx_ref[...] = (b * (_L * 4096) + nbr_t * 4096
                        + (ctr >> 7) * 512 + (ctr & 127))
    rdsum_ref[0] = rdsum


def _knn(pos, pos_t):
    return pl.pallas_call(
        _knn_body,
        grid=(_B, _L // _RB),
        in_specs=[
            pl.BlockSpec((1, _RB, 3), lambda b, r: (b, r, 0)),
            pl.BlockSpec((1, 3, _L), lambda b, r: (b, 0, 0)),
        ],
        out_specs=[
            pl.BlockSpec((32, _RB), lambda b, r: (b, r)),
            pl.BlockSpec((1, _RB, 1), lambda b, r: (b, r, 0)),
        ],
        out_shape=[
            jax.ShapeDtypeStruct((_B * 32, _L), jnp.int32),
            jax.ShapeDtypeStruct((_B, _L, 1), jnp.float32),
        ],
    )(pos, pos_t)


_NC, _NS = 2, 16
_NW = _NC * _NS
_NODES_PER = _N // _NW
_CHUNKS = _NODES_PER * _K // 128


_EPT = _NODES_PER * _K
_QTOT = 4 * _CHUNKS
_QBATCH = 24


def _gather_body(table_hbm, idx_hbm, out_hbm, idx_v, idx4_v, ch_v, agg_v, sem):
    wid = lax.axis_index("s") * _NC + lax.axis_index("c")
    pltpu.sync_copy(
        idx_hbm.at[pl.ds((wid >> 3) * 32, _CHUNKS), pl.ds((wid & 7) * 128, 128)],
        idx_v)

    def body_idx(k, carry):
        for i in range(8):
            v = idx_v[k, pl.ds(i * 16, 16)]
            for c in range(4):
                idx4_v[c * _CHUNKS + k, pl.ds(i * 16, 16)] = v + c * 128
        return carry

    lax.fori_loop(0, _CHUNKS, body_idx, 0)

    def fire(q):
        pltpu.async_copy(
            table_hbm.at[idx4_v.at[q]], ch_v.at[pl.ds(q * 128, 128)], sem)

    def drain(q):
        pltpu.make_async_copy(
            table_hbm.at[idx4_v.at[q]], ch_v.at[pl.ds(q * 128, 128)], sem).wait()

    nb = _QTOT // _QBATCH
    for i in range(nb):
        for q in range(i * _QBATCH, (i + 1) * _QBATCH):
            fire(q)
        if i > 0:
            for q in range((i - 1) * _QBATCH, i * _QBATCH):
                drain(q)
    for q in range((nb - 1) * _QBATCH, nb * _QBATCH):
        drain(q)

    inv_k = jnp.full((16,), 1.0 / _K, jnp.float32)

    def body(g, carry):
        for c in range(4):
            acc = jnp.zeros((16,), jnp.float32)
            for k in range(_K):
                acc = acc + ch_v[pl.ds(c * _EPT + k * 128 + g * 16, 16)]
            agg_v[pl.ds(c * _NODES_PER + g * 16, 16)] = acc * inv_k
        return carry

    lax.fori_loop(0, _NODES_PER // 16, body, 0)
    for c in range(4):
        pltpu.sync_copy(
            agg_v.at[pl.ds(c * _NODES_PER, _NODES_PER)],
            out_hbm.at[pl.ds(c * _N + wid * _NODES_PER, _NODES_PER)])


@functools.cache
def _gather_call():
    return pl.kernel(
        _gather_body,
        out_type=jax.ShapeDtypeStruct((4 * _N,), jnp.float32),
        mesh=plsc.VectorSubcoreMesh(
            core_axis_name="c", subcore_axis_name="s",
            num_cores=_NC, num_subcores=_NS,
        ),
        scratch_types=[
            pltpu.VMEM((_CHUNKS, 128), jnp.int32),
            pltpu.VMEM((_QTOT, 128), jnp.int32),
            pltpu.VMEM((_QTOT * 128,), jnp.float32),
            pltpu.VMEM((4 * _NODES_PER,), jnp.float32),
            pltpu.SemaphoreType.DMA,
        ],
        compiler_params=pltpu.CompilerParams(
            use_tc_tiling_on_sc=False, needs_layout_passes=False),
    )


_RM = 512


def _mlp_body(x_ref, ap_ref, rs_ref, *refs):
    w1s, b1s, w2s, b2s = refs[0:3], refs[3:6], refs[6:9], refs[9:12]
    out_ref = refs[12]
    x = x_ref[...]
    ap = ap_ref[...]
    rs = rs_ref[...] * jnp.float32(1.0 / _K)
    for l in range(3):
        w1 = w1s[l]
        h = jnp.dot(x, w1[0:_D, :], preferred_element_type=jnp.float32)
        for c in range(4):
            h = h + ap[:, c:c + 1] * w1[_D + c:_D + c + 1, :]
        h = h + rs * w1[_D + 4:_D + 5, :] + b1s[l][...]
        h = jnp.maximum(h, 0.0)
        x = jnp.dot(h, w2s[l][...], preferred_element_type=jnp.float32) + b2s[l][...]
    out_ref[...] = x


def _mlp(x, aggp, rdsum, w1s, b1s, w2s, b2s):
    full = lambda shape: pl.BlockSpec(shape, lambda i: tuple(0 for _ in shape))
    return pl.pallas_call(
        _mlp_body,
        grid=(_N // _RM,),
        in_specs=[
            pl.BlockSpec((_RM, _D), lambda i: (i, 0)),
            pl.BlockSpec((_RM, 4), lambda i: (i, 0)),
            pl.BlockSpec((_RM, 1), lambda i: (i, 0)),
            *[full((_D + 5, _D)) for _ in range(3)],
            *[full((1, _D)) for _ in range(3)],
            *[full((_D, _D)) for _ in range(3)],
            *[full((1, _D)) for _ in range(3)],
        ],
        out_specs=pl.BlockSpec((_RM, _D), lambda i: (i, 0)),
        out_shape=jax.ShapeDtypeStruct((_N, _D), jnp.float32),
    )(x, aggp, rdsum, *w1s, *b1s, *w2s, *b2s)


def kernel(R, t, res_feat, pair_feat, mask_res, params):
    pos = t
    pos_t = jnp.swapaxes(t, 1, 2)
    pairidx, rdsum = _knn(pos, pos_t)
    tflat = (pair_feat.reshape(_B, _L, _L // 128, 128, 4)
             .transpose(0, 1, 2, 4, 3).reshape(4 * _B * _L * _L))
    aggp = _gather_call()(tflat, pairidx).reshape(4, _N).T
    w1s = [p["nw1"] for p in params]
    b1s = [p["nb1"].reshape(1, _D) for p in params]
    w2s = [p["nw2"] for p in params]
    b2s = [p["nb2"].reshape(1, _D) for p in params]
    x = res_feat.reshape(_N, _D)
    out = _mlp(x, aggp, rdsum.reshape(_N, 1), w1s, b1s, w2s, b2s)
    return out.reshape(_B, _L, _D)


# speedup vs baseline: 16.2857x; 1.0051x over previous
import functools

import jax
import jax.numpy as jnp
from jax import lax
from jax.experimental import pallas as pl
from jax.experimental.pallas import tpu as pltpu
from jax.experimental.pallas import tpu_sc as plsc

_B, _L, _K = 4, 1024, 30
_D = 128
_N = _B * _L
_E = _N * _K


_RB = 256


_NCH = 8


def _knn_body(pos_r_ref, pos_j_ref, pairidx_ref, rdsum_ref):
    b = pl.program_id(0)
    r = pl.program_id(1)
    pr = pos_r_ref[0]
    pj = pos_j_ref[0]
    jiof = lax.broadcasted_iota(jnp.int32, (_RB, _L), 1).astype(jnp.float32)
    rowgf = (lax.broadcasted_iota(jnp.int32, (_RB, _L), 0) + r * _RB
             ).astype(jnp.float32)
    acc = (pr[:, 0:1] - pj[0:1, :]) ** 2
    acc = acc + (pr[:, 1:2] - pj[1:2, :]) ** 2
    acc = acc + (pr[:, 2:3] - pj[2:3, :]) ** 2
    d2 = jnp.where(jiof == rowgf, jnp.float32(1e9), acc)
    big = jnp.float32(3e38)
    cols = []
    rdsum = jnp.zeros((_RB, 1), jnp.float32)
    idxf = None
    for _ in range(_K):
        if idxf is not None:
            d2 = jnp.where(jiof == idxf, big, d2)
        m = jnp.min(d2, axis=1, keepdims=True)
        idxf = jnp.min(jnp.where(d2 == m, jiof, jnp.float32(4096.0)),
                       axis=1, keepdims=True)
        rdsum = rdsum + jnp.sqrt(m)
        cols.append(idxf)
    nbr = jnp.concatenate(cols, axis=1).astype(jnp.int32)
    nbr_t = jnp.transpose(nbr)
    nbr_t = jnp.concatenate([nbr_t, nbr_t[0:2]], axis=0)
    ctr = lax.broadcasted_iota(jnp.int32, (32, _RB), 1) + r * _RB
    pairidx_ref[...] = (b * (_L * 4096) + nbr_t * 4096
                        + (ctr >> 7) * 512 + (ctr & 127))
    rdsum_ref[...] = rdsum


def _knn(pos, pos_t):
    return pl.pallas_call(
        _knn_body,
        grid=(_B, _L // _RB),
        in_specs=[
            pl.BlockSpec((1, _RB, 3), lambda b, r: (b, r, 0)),
            pl.BlockSpec((1, 3, _L), lambda b, r: (b, 0, 0)),
        ],
        out_specs=[
            pl.BlockSpec((32, _RB), lambda b, r: (b, r)),
            pl.BlockSpec((_RB, 1), lambda b, r: (b * (_L // _RB) + r, 0)),
        ],
        out_shape=[
            jax.ShapeDtypeStruct((_B * 32, _L), jnp.int32),
            jax.ShapeDtypeStruct((_N, 1), jnp.float32),
        ],
    )(pos, pos_t)


_NC, _NS = 2, 16
_NW = _NC * _NS
_NODES_PER = _N // _NW
_CHUNKS = _NODES_PER * _K // 128


_EPT = _NODES_PER * _K
_QTOT = 4 * _CHUNKS
_QBATCH = 24


def _gather_body(table_hbm, idx_hbm, out_hbm, idx_v, idx4_v, ch_v, agg_v, sem):
    wid = lax.axis_index("s") * _NC + lax.axis_index("c")
    pltpu.sync_copy(
        idx_hbm.at[pl.ds((wid >> 3) * 32, _CHUNKS), pl.ds((wid & 7) * 128, 128)],
        idx_v)

    def body_idx(k, carry):
        for i in range(8):
            v = idx_v[k, pl.ds(i * 16, 16)]
            for c in range(4):
                idx4_v[c * _CHUNKS + k, pl.ds(i * 16, 16)] = v + c * 128
        return carry

    lax.fori_loop(0, _CHUNKS, body_idx, 0)

    def fire(q):
        pltpu.async_copy(
            table_hbm.at[idx4_v.at[q]], ch_v.at[pl.ds(q * 128, 128)], sem)

    def drain(q):
        pltpu.make_async_copy(
            table_hbm.at[idx4_v.at[q]], ch_v.at[pl.ds(q * 128, 128)], sem).wait()

    nb = _QTOT // _QBATCH
    for i in range(nb):
        for q in range(i * _QBATCH, (i + 1) * _QBATCH):
            fire(q)
        if i > 0:
            for q in range((i - 1) * _QBATCH, i * _QBATCH):
                drain(q)
    for q in range((nb - 1) * _QBATCH, nb * _QBATCH):
        drain(q)

    inv_k = jnp.full((16,), 1.0 / _K, jnp.float32)

    def body(g, carry):
        for c in range(4):
            acc = jnp.zeros((16,), jnp.float32)
            for k in range(_K):
                acc = acc + ch_v[pl.ds(c * _EPT + k * 128 + g * 16, 16)]
            agg_v[pl.ds(c * _NODES_PER + g * 16, 16)] = acc * inv_k
        return carry

    lax.fori_loop(0, _NODES_PER // 16, body, 0)
    for c in range(4):
        pltpu.sync_copy(
            agg_v.at[pl.ds(c * _NODES_PER, _NODES_PER)],
            out_hbm.at[pl.ds(c * _N + wid * _NODES_PER, _NODES_PER)])


@functools.cache
def _gather_call():
    return pl.kernel(
        _gather_body,
        out_type=jax.ShapeDtypeStruct((4 * _N,), jnp.float32),
        mesh=plsc.VectorSubcoreMesh(
            core_axis_name="c", subcore_axis_name="s",
            num_cores=_NC, num_subcores=_NS,
        ),
        scratch_types=[
            pltpu.VMEM((_CHUNKS, 128), jnp.int32),
            pltpu.VMEM((_QTOT, 128), jnp.int32),
            pltpu.VMEM((_QTOT * 128,), jnp.float32),
            pltpu.VMEM((4 * _NODES_PER,), jnp.float32),
            pltpu.SemaphoreType.DMA,
        ],
        compiler_params=pltpu.CompilerParams(
            use_tc_tiling_on_sc=False, needs_layout_passes=False),
    )


_RM = 512


def _mlp_body(x_ref, ap_ref, rs_ref, *refs):
    w1s, b1s, w2s, b2s = refs[0:3], refs[3:6], refs[6:9], refs[9:12]
    out_ref = refs[12]
    x = x_ref[...]
    ap = jnp.transpose(ap_ref[...])
    rs = rs_ref[...] * jnp.float32(1.0 / _K)
    for l in range(3):
        w1 = w1s[l]
        h = jnp.dot(x, w1[0:_D, :], preferred_element_type=jnp.float32)
        for c in range(4):
            h = h + ap[:, c:c + 1] * w1[_D + c:_D + c + 1, :]
        h = h + rs * w1[_D + 4:_D + 5, :] + b1s[l][...]
        h = jnp.maximum(h, 0.0)
        x = jnp.dot(h, w2s[l][...], preferred_element_type=jnp.float32) + b2s[l][...]
    out_ref[...] = x


def _mlp(x, aggp, rdsum, w1s, b1s, w2s, b2s):
    full = lambda shape: pl.BlockSpec(shape, lambda i: tuple(0 for _ in shape))
    return pl.pallas_call(
        _mlp_body,
        grid=(_N // _RM,),
        in_specs=[
            pl.BlockSpec((_RM, _D), lambda i: (i, 0)),
            pl.BlockSpec((4, _RM), lambda i: (0, i)),
            pl.BlockSpec((_RM, 1), lambda i: (i, 0)),
            *[full((_D + 5, _D)) for _ in range(3)],
            *[full((1, _D)) for _ in range(3)],
            *[full((_D, _D)) for _ in range(3)],
            *[full((1, _D)) for _ in range(3)],
        ],
        out_specs=pl.BlockSpec((_RM, _D), lambda i: (i, 0)),
        out_shape=jax.ShapeDtypeStruct((_N, _D), jnp.float32),
    )(x, aggp, rdsum, *w1s, *b1s, *w2s, *b2s)


def kernel(R, t, res_feat, pair_feat, mask_res, params):
    pos = t
    pos_t = jnp.swapaxes(t, 1, 2)
    pairidx, rdsum = _knn(pos, pos_t)
    tflat = (pair_feat.reshape(_B, _L, _L // 128, 128, 4)
             .transpose(0, 1, 2, 4, 3).reshape(4 * _B * _L * _L))
    aggp_t = _gather_call()(tflat, pairidx).reshape(4, _N)
    w1s = [p["nw1"] for p in params]
    b1s = [p["nb1"].reshape(1, _D) for p in params]
    w2s = [p["nw2"] for p in params]
    b2s = [p["nb2"].reshape(1, _D) for p in params]
    x = res_feat.reshape(_N, _D)
    out = _mlp(x, aggp_t, rdsum, w1s, b1s, w2s, b2s)
    return out.reshape(_B, _L, _D)


# speedup vs baseline: 16.8712x; 1.0360x over previous
import functools

import jax
import jax.numpy as jnp
from jax import lax
from jax.experimental import pallas as pl
from jax.experimental.pallas import tpu as pltpu
from jax.experimental.pallas import tpu_sc as plsc

_B, _L, _K = 4, 1024, 30
_D = 128
_N = _B * _L
_E = _N * _K


_RB = 256


_NCH = 8


def _knn_body(b0, pos_r_ref, pos_j_ref, pairidx_ref, rdsum_ref):
    b = pl.program_id(0) + b0
    r = pl.program_id(1)
    pr = pos_r_ref[0]
    pj = pos_j_ref[0]
    jiof = lax.broadcasted_iota(jnp.int32, (_RB, _L), 1).astype(jnp.float32)
    rowgf = (lax.broadcasted_iota(jnp.int32, (_RB, _L), 0) + r * _RB
             ).astype(jnp.float32)
    acc = (pr[:, 0:1] - pj[0:1, :]) ** 2
    acc = acc + (pr[:, 1:2] - pj[1:2, :]) ** 2
    acc = acc + (pr[:, 2:3] - pj[2:3, :]) ** 2
    d2 = jnp.where(jiof == rowgf, jnp.float32(1e9), acc)
    big = jnp.float32(3e38)
    cols = []
    rdsum = jnp.zeros((_RB, 1), jnp.float32)
    idxf = None
    for _ in range(_K):
        if idxf is not None:
            d2 = jnp.where(jiof == idxf, big, d2)
        m = jnp.min(d2, axis=1, keepdims=True)
        idxf = jnp.min(jnp.where(d2 == m, jiof, jnp.float32(4096.0)),
                       axis=1, keepdims=True)
        rdsum = rdsum + jnp.sqrt(m)
        cols.append(idxf)
    nbr = jnp.concatenate(cols, axis=1).astype(jnp.int32)
    nbr_t = jnp.transpose(nbr)
    nbr_t = jnp.concatenate([nbr_t, nbr_t[0:2]], axis=0)
    ctr = lax.broadcasted_iota(jnp.int32, (32, _RB), 1) + r * _RB
    pairidx_ref[...] = (b * (_L * 4096) + nbr_t * 4096
                        + (ctr >> 7) * 512 + (ctr & 127))
    rdsum_ref[...] = rdsum


_HB = 2
_HN = _HB * _L


def _knn_half(pos, pos_t, b0):
    return pl.pallas_call(
        functools.partial(_knn_body, b0),
        grid=(_HB, _L // _RB),
        in_specs=[
            pl.BlockSpec((1, _RB, 3), lambda b, r: (b, r, 0)),
            pl.BlockSpec((1, 3, _L), lambda b, r: (b, 0, 0)),
        ],
        out_specs=[
            pl.BlockSpec((32, _RB), lambda b, r: (b, r)),
            pl.BlockSpec((_RB, 1), lambda b, r: (b * (_L // _RB) + r, 0)),
        ],
        out_shape=[
            jax.ShapeDtypeStruct((_HB * 32, _L), jnp.int32),
            jax.ShapeDtypeStruct((_HN, 1), jnp.float32),
        ],
    )(pos, pos_t)


_NC, _NS = 2, 16
_NW = _NC * _NS
_NODES_PER = _HN // _NW
_EPT = _NODES_PER * _K
_CHUNKS = _EPT // 128
_QTOT = 4 * _CHUNKS
_QBATCH = 20


def _gather_body(table_hbm, idx_hbm, out_hbm, idx_v, idx4_v, ch_v, agg_v, sem):
    wid = lax.axis_index("s") * _NC + lax.axis_index("c")
    pltpu.sync_copy(
        idx_hbm.at[pl.ds((wid >> 4) * 32, _K), pl.ds((wid & 15) * 64, 64)],
        idx_v)

    def body_idx(k2, carry):
        for half in range(2):
            for i in range(4):
                v = idx_v[2 * k2 + half, pl.ds(i * 16, 16)]
                for c in range(4):
                    idx4_v[c * _CHUNKS + k2,
                           pl.ds(half * 64 + i * 16, 16)] = v + c * 128
        return carry

    lax.fori_loop(0, _CHUNKS, body_idx, 0)

    def fire(q):
        pltpu.async_copy(
            table_hbm.at[idx4_v.at[q]], ch_v.at[pl.ds(q * 128, 128)], sem)

    def drain(q):
        pltpu.make_async_copy(
            table_hbm.at[idx4_v.at[q]], ch_v.at[pl.ds(q * 128, 128)], sem).wait()

    nb = _QTOT // _QBATCH
    for i in range(nb):
        for q in range(i * _QBATCH, (i + 1) * _QBATCH):
            fire(q)
        if i > 0:
            for q in range((i - 1) * _QBATCH, i * _QBATCH):
                drain(q)
    for q in range((nb - 1) * _QBATCH, nb * _QBATCH):
        drain(q)

    inv_k = jnp.full((16,), 1.0 / _K, jnp.float32)

    def body(g, carry):
        for c in range(4):
            acc = jnp.zeros((16,), jnp.float32)
            for k in range(_K):
                acc = acc + ch_v[pl.ds(c * _EPT + k * 64 + g * 16, 16)]
            agg_v[pl.ds(c * _NODES_PER + g * 16, 16)] = acc * inv_k
        return carry

    lax.fori_loop(0, _NODES_PER // 16, body, 0)
    for c in range(4):
        pltpu.sync_copy(
            agg_v.at[pl.ds(c * _NODES_PER, _NODES_PER)],
            out_hbm.at[pl.ds(c * _HN + wid * _NODES_PER, _NODES_PER)])


@functools.cache
def _gather_call():
    return pl.kernel(
        _gather_body,
        out_type=jax.ShapeDtypeStruct((4 * _HN,), jnp.float32),
        mesh=plsc.VectorSubcoreMesh(
            core_axis_name="c", subcore_axis_name="s",
            num_cores=_NC, num_subcores=_NS,
        ),
        scratch_types=[
            pltpu.VMEM((_K, 64), jnp.int32),
            pltpu.VMEM((_QTOT, 128), jnp.int32),
            pltpu.VMEM((_QTOT * 128,), jnp.float32),
            pltpu.VMEM((4 * _NODES_PER,), jnp.float32),
            pltpu.SemaphoreType.DMA,
        ],
        compiler_params=pltpu.CompilerParams(
            use_tc_tiling_on_sc=False, needs_layout_passes=False),
    )


_RM = 512


def _mlp_body(x_ref, ap_ref, rs_ref, *refs):
    w1s, b1s, w2s, b2s = refs[0:3], refs[3:6], refs[6:9], refs[9:12]
    out_ref = refs[12]
    x = x_ref[...]
    ap = jnp.transpose(ap_ref[...])
    rs = rs_ref[...] * jnp.float32(1.0 / _K)
    for l in range(3):
        w1 = w1s[l]
        h = jnp.dot(x, w1[0:_D, :], preferred_element_type=jnp.float32)
        for c in range(4):
            h = h + ap[:, c:c + 1] * w1[_D + c:_D + c + 1, :]
        h = h + rs * w1[_D + 4:_D + 5, :] + b1s[l][...]
        h = jnp.maximum(h, 0.0)
        x = jnp.dot(h, w2s[l][...], preferred_element_type=jnp.float32) + b2s[l][...]
    out_ref[...] = x


def _mlp(x, aggp, rdsum, w1s, b1s, w2s, b2s):
    full = lambda shape: pl.BlockSpec(shape, lambda i: tuple(0 for _ in shape))
    return pl.pallas_call(
        _mlp_body,
        grid=(_N // _RM,),
        in_specs=[
            pl.BlockSpec((_RM, _D), lambda i: (i, 0)),
            pl.BlockSpec((4, _RM), lambda i: (0, i)),
            pl.BlockSpec((_RM, 1), lambda i: (i, 0)),
            *[full((_D + 5, _D)) for _ in range(3)],
            *[full((1, _D)) for _ in range(3)],
            *[full((_D, _D)) for _ in range(3)],
            *[full((1, _D)) for _ in range(3)],
        ],
        out_specs=pl.BlockSpec((_RM, _D), lambda i: (i, 0)),
        out_shape=jax.ShapeDtypeStruct((_N, _D), jnp.float32),
    )(x, aggp, rdsum, *w1s, *b1s, *w2s, *b2s)


def kernel(R, t, res_feat, pair_feat, mask_res, params):
    pos = t
    pos_t = jnp.swapaxes(t, 1, 2)
    tflat = (pair_feat.reshape(_B, _L, _L // 128, 128, 4)
             .transpose(0, 1, 2, 4, 3).reshape(4 * _B * _L * _L))
    gather = _gather_call()
    pairidx0, rdsum0 = _knn_half(pos[0:_HB], pos_t[0:_HB], 0)
    pairidx1, rdsum1 = _knn_half(pos[_HB:], pos_t[_HB:], _HB)
    agg0 = gather(tflat, pairidx0).reshape(4, _HN)
    agg1 = gather(tflat, pairidx1).reshape(4, _HN)
    aggp_t = jnp.concatenate([agg0, agg1], axis=1)
    rdsum = jnp.concatenate([rdsum0, rdsum1], axis=0)
    w1s = [p["nw1"] for p in params]
    b1s = [p["nb1"].reshape(1, _D) for p in params]
    w2s = [p["nw2"] for p in params]
    b2s = [p["nb2"].reshape(1, _D) for p in params]
    x = res_feat.reshape(_N, _D)
    out = _mlp(x, aggp_t, rdsum, w1s, b1s, w2s, b2s)
    return out.reshape(_B, _L, _D)
